# baseline (speedup 1.0000x reference)
import functools

import jax
import jax.numpy as jnp
from jax import lax
from jax.experimental import pallas as pl
from jax.experimental.pallas import tpu as pltpu
from jax.experimental.pallas import tpu_sc as plsc

_UNROLL = 5


def _sc_gather(table, idx, *, bq, rows_b, feat, n_chunks, chunk):
    mesh = plsc.VectorSubcoreMesh(core_axis_name="c", subcore_axis_name="s")
    info = plsc.get_sparse_core_info()
    nc = info.num_cores
    rows_w = n_chunks * chunk

    @functools.partial(
        pl.kernel,
        mesh=mesh,
        out_type=jax.ShapeDtypeStruct((bq, rows_b, feat), jnp.int32),
        compiler_params=pltpu.CompilerParams(use_tc_tiling_on_sc=False),
        scratch_types=[
            pltpu.VMEM((n_chunks, chunk), jnp.int32),
            pltpu.VMEM((_UNROLL, chunk, feat), jnp.int32),
        ] + [pltpu.SemaphoreType.DMA] * _UNROLL,
    )
    def gather_kernel(table_hbm, idx_hbm, out_hbm, idx_v, rows_v, *sems):
        cid = lax.axis_index("c")
        sid = lax.axis_index("s")
        wid = sid * nc + cid
        row0 = sid * rows_w
        pltpu.sync_copy(idx_hbm.at[wid], idx_v)

        def body(p, _):
            c0 = p * _UNROLL
            handles = [
                pltpu.async_copy(table_hbm.at[idx_v.at[c0 + k]],
                                 rows_v.at[k], sems[k])
                for k in range(_UNROLL)
            ]
            for k in range(_UNROLL):
                handles[k].wait()
                pltpu.sync_copy(
                    rows_v.at[k],
                    out_hbm.at[cid, pl.ds(row0 + (c0 + k) * chunk, chunk)])
            return _

        lax.fori_loop(0, n_chunks // _UNROLL, body, None)

    return gather_kernel(table, idx)


def _tc_body(m_edges, x_ref, g_ref, e_ref, ws_ref, wne_ref, wno_ref, we_ref,
             b_ref, alpha_ref, o_ref):
    tn = x_ref.shape[1]
    x = x_ref[0]
    gp = g_ref[0]
    e = e_ref[0]
    g_even = lax.bitcast_convert_type(
        lax.shift_left(gp, 16), jnp.float32).astype(jnp.bfloat16)
    g_odd = lax.bitcast_convert_type(
        jnp.bitwise_and(gp, jnp.int32(-65536)), jnp.float32).astype(jnp.bfloat16)
    ps = jnp.dot(x, ws_ref[...], preferred_element_type=jnp.float32)
    ps = ps + b_ref[...]
    pg = (jnp.dot(g_even, wne_ref[...], preferred_element_type=jnp.float32)
          + jnp.dot(g_odd, wno_ref[...], preferred_element_type=jnp.float32))
    pe = jnp.dot(e, we_ref[...], preferred_element_type=jnp.float32)
    gated = (pg + pe).reshape(tn, m_edges, ps.shape[-1]) + ps[:, None, :]
    half = ps.shape[-1] // 2
    filt_x = gated[..., :half]
    core_x = gated[..., half:]
    filt = 1.0 / (1.0 + jnp.exp(-filt_x))
    core = jnp.maximum(core_x, 0.0) + jnp.log1p(jnp.exp(-jnp.abs(core_x)))
    s = jnp.sum(filt * core, axis=1)
    z = alpha_ref[0, 0] * x + s
    o_ref[0] = jnp.maximum(z, 0.0) + jnp.log1p(jnp.exp(-jnp.abs(z)))


def _tc_fused(node, gathered, edge, ws, wne, wno, we, bvec, alpha, *, tn):
    bq, nq, d = node.shape
    m_edges = gathered.shape[1] // nq
    ef = edge.shape[-1]
    dd = ws.shape[-1]
    dp = gathered.shape[-1]
    grid = (bq, nq // tn)
    return pl.pallas_call(
        functools.partial(_tc_body, m_edges),
        grid=grid,
        in_specs=[
            pl.BlockSpec((1, tn, d), lambda b, i: (b, i, 0)),
            pl.BlockSpec((1, tn * m_edges, dp), lambda b, i: (b, i, 0)),
            pl.BlockSpec((1, tn * m_edges, ef), lambda b, i: (b, i, 0)),
            pl.BlockSpec((d, dd), lambda b, i: (0, 0)),
            pl.BlockSpec((d // 2, dd), lambda b, i: (0, 0)),
            pl.BlockSpec((d // 2, dd), lambda b, i: (0, 0)),
            pl.BlockSpec((ef, dd), lambda b, i: (0, 0)),
            pl.BlockSpec((1, dd), lambda b, i: (0, 0)),
            pl.BlockSpec(memory_space=pltpu.SMEM),
        ],
        out_specs=pl.BlockSpec((1, tn, d), lambda b, i: (b, i, 0)),
        out_shape=jax.ShapeDtypeStruct((bq, nq, d), jnp.float32),
    )(node, gathered, edge, ws, wne, wno, we, bvec, alpha)


def kernel(node_in_fea, edge_fea, edge_fea_idx, W, b, alpha):
    bq, nq, mq = edge_fea_idx.shape
    d = node_in_fea.shape[-1]
    ef = edge_fea.shape[-1]

    info = plsc.get_sparse_core_info()
    nc, ns = info.num_cores, info.num_subcores
    n_workers = nc * ns
    rows_b = nq * mq
    chunk = 80
    per_worker = (bq * rows_b) // n_workers
    n_chunks = per_worker // chunk
    assert bq == nc and per_worker == n_chunks * chunk
    assert n_chunks % _UNROLL == 0

    node_bf = node_in_fea.astype(jnp.bfloat16)
    table = lax.bitcast_convert_type(
        node_bf.reshape(bq * nq, d // 2, 2), jnp.int32)

    offs = (jnp.arange(bq, dtype=jnp.int32) * nq)[:, None]
    flat_idx = edge_fea_idx.astype(jnp.int32).reshape(bq, rows_b) + offs
    idx_arr = (flat_idx.reshape(bq, ns, n_chunks, chunk)
               .transpose(1, 0, 2, 3).reshape(n_workers, n_chunks, chunk))

    gathered = _sc_gather(table, idx_arr, bq=bq, rows_b=rows_b, feat=d // 2,
                          n_chunks=n_chunks, chunk=chunk)

    ws = W[:, :d].T
    wn = W[:, d:2 * d].T
    wne = wn[0::2].astype(jnp.bfloat16)
    wno = wn[1::2].astype(jnp.bfloat16)
    we = W[:, 2 * d:].T
    bvec = b.reshape(1, -1)
    alpha2 = jnp.asarray(alpha, jnp.float32).reshape(1, 1)
    edge2 = edge_fea.reshape(bq, rows_b, ef)

    return _tc_fused(node_in_fea, gathered, edge2, ws, wne, wno, we, bvec,
                     alpha2, tn=200)


# speedup vs baseline: 1.4964x; 1.4964x over previous
import functools

import jax
import jax.numpy as jnp
from jax import lax
from jax.experimental import pallas as pl
from jax.experimental.pallas import tpu as pltpu
from jax.experimental.pallas import tpu_sc as plsc

_UNROLL = 5
_SLICES = 5


def _sc_gather(table, idx, *, rows, feat, n_chunks, chunk):
    mesh = plsc.VectorSubcoreMesh(core_axis_name="c", subcore_axis_name="s")
    info = plsc.get_sparse_core_info()
    nc = info.num_cores
    rows_w = n_chunks * chunk

    @functools.partial(
        pl.kernel,
        mesh=mesh,
        out_type=jax.ShapeDtypeStruct((rows, feat), jnp.float32),
        scratch_types=[
            pltpu.VMEM((n_chunks, chunk), jnp.int32),
            pltpu.VMEM((_UNROLL, chunk, feat), jnp.float32),
        ] + [pltpu.SemaphoreType.DMA] * _UNROLL,
    )
    def gather_kernel(table_hbm, idx_hbm, out_hbm, idx_v, rows_v, *sems):
        wid = lax.axis_index("s") * nc + lax.axis_index("c")
        row0 = wid * rows_w
        pltpu.sync_copy(idx_hbm.at[wid], idx_v)

        def body(p, _):
            c0 = p * _UNROLL
            handles = [
                pltpu.async_copy(table_hbm.at[idx_v.at[c0 + k]],
                                 rows_v.at[k], sems[k])
                for k in range(_UNROLL)
            ]
            for k in range(_UNROLL):
                handles[k].wait()
                pltpu.sync_copy(
                    rows_v.at[k],
                    out_hbm.at[pl.ds(row0 + (c0 + k) * chunk, chunk)])
            return _

        lax.fori_loop(0, n_chunks // _UNROLL, body, None)

    return gather_kernel(table, idx)


def _tc_body(m_edges, x_ref, g_ref, e_ref, ws_ref, wn_ref, we_ref,
             b_ref, alpha_ref, o_ref):
    tn = x_ref.shape[1]
    x = x_ref[0]
    g = g_ref[...]
    e = e_ref[0]
    ps = jnp.dot(x, ws_ref[...], preferred_element_type=jnp.float32)
    ps = ps + b_ref[...]
    pg = jnp.dot(g.astype(jnp.bfloat16), wn_ref[...],
                 preferred_element_type=jnp.float32)
    pe = jnp.dot(e, we_ref[...], preferred_element_type=jnp.float32)
    gated = (pg + pe).reshape(tn, m_edges, ps.shape[-1]) + ps[:, None, :]
    half = ps.shape[-1] // 2
    filt_x = gated[..., :half]
    core_x = gated[..., half:]
    filt = 1.0 / (1.0 + jnp.exp(-filt_x))
    core = jnp.maximum(core_x, 0.0) + jnp.log1p(jnp.exp(-jnp.abs(core_x)))
    s = jnp.sum(filt * core, axis=1)
    z = alpha_ref[0, 0] * x + s
    o_ref[...] = jnp.maximum(z, 0.0) + jnp.log1p(jnp.exp(-jnp.abs(z)))


def _tc_fused(node, gathered, edge, ws, wn, we, bvec, alpha, *, tn, bi, s,
              nodes_s):
    bq, nq, d = node.shape
    mq = edge.shape[1] // nq
    m_edges = mq
    ef = edge.shape[-1]
    dd = ws.shape[-1]
    nblk = nodes_s // tn
    grid = (nblk,)
    return pl.pallas_call(
        functools.partial(_tc_body, m_edges),
        grid=grid,
        in_specs=[
            pl.BlockSpec((1, tn, d), lambda i: (bi, s * nblk + i, 0)),
            pl.BlockSpec((tn * m_edges, d), lambda i: (i, 0)),
            pl.BlockSpec((1, tn * m_edges, ef), lambda i: (bi, s * nblk + i, 0)),
            pl.BlockSpec((d, dd), lambda i: (0, 0)),
            pl.BlockSpec((d, dd), lambda i: (0, 0)),
            pl.BlockSpec((ef, dd), lambda i: (0, 0)),
            pl.BlockSpec((1, dd), lambda i: (0, 0)),
            pl.BlockSpec(memory_space=pltpu.SMEM),
        ],
        out_specs=pl.BlockSpec((tn, d), lambda i: (i, 0)),
        out_shape=jax.ShapeDtypeStruct((nodes_s, d), jnp.float32),
    )(node, gathered, edge, ws, wn, we, bvec, alpha)


def kernel(node_in_fea, edge_fea, edge_fea_idx, W, b, alpha):
    bq, nq, mq = edge_fea_idx.shape
    d = node_in_fea.shape[-1]
    ef = edge_fea.shape[-1]

    info = plsc.get_sparse_core_info()
    n_workers = info.num_cores * info.num_subcores
    nodes_s = nq // _SLICES
    rows_s = nodes_s * mq
    chunk = 40
    per_worker = rows_s // n_workers
    n_chunks = per_worker // chunk
    assert per_worker == n_chunks * chunk and n_chunks % _UNROLL == 0
    assert nq == _SLICES * nodes_s

    table = node_in_fea.reshape(bq * nq, d)

    ws = W[:, :d].T
    wn = W[:, d:2 * d].T.astype(jnp.bfloat16)
    we = W[:, 2 * d:].T
    bvec = b.reshape(1, -1)
    alpha2 = jnp.asarray(alpha, jnp.float32).reshape(1, 1)
    edge3 = edge_fea.reshape(bq, nq * mq, ef)

    gathers = []
    for bi in range(bq):
        for s in range(_SLICES):
            idx_s = (edge_fea_idx[bi, s * nodes_s:(s + 1) * nodes_s]
                     .astype(jnp.int32).reshape(n_workers, n_chunks, chunk)
                     + bi * nq)
            gathers.append(_sc_gather(table, idx_s, rows=rows_s, feat=d,
                                      n_chunks=n_chunks, chunk=chunk))

    outs = []
    for bi in range(bq):
        for s in range(_SLICES):
            outs.append(_tc_fused(node_in_fea, gathers[bi * _SLICES + s],
                                  edge3, ws, wn, we, bvec, alpha2,
                                  tn=200, bi=bi, s=s, nodes_s=nodes_s))

    return jnp.stack([jnp.concatenate(outs[bi * _SLICES:(bi + 1) * _SLICES])
                      for bi in range(bq)])
